# Initial kernel scaffold; baseline (speedup 1.0000x reference)
import jax
import jax.numpy as jnp
from jax.experimental import pallas as pl


def kernel(x, up_index, up_attr, down_index, down_attr, boundary_index, params):
    raise NotImplementedError("write your pallas kernel here")


# speedup vs baseline: 2.0253x; 2.0253x over previous
import functools

import jax
import jax.numpy as jnp
from jax import lax
from jax.experimental import pallas as pl
from jax.experimental.pallas import tpu as pltpu
from jax.experimental.pallas import tpu_sc as plsc

N = 10000
D = 128
H = 128
E_UP = 320000
E_DOWN = 160000
E_B = 160000

NC = 2
NS = 16
L = 16
C = 80
RPT = 624
RTAIL = N - RPT * NS


def _prep_body(x_ref, au_ref, bu_ref, ad_ref, bd_ref, yu_ref, yd_ref):
    x = x_ref[...]
    yu_ref[...] = jnp.dot(x, au_ref[...], preferred_element_type=jnp.float32) + bu_ref[...]
    yd_ref[...] = jnp.dot(x, ad_ref[...], preferred_element_type=jnp.float32) + bd_ref[...]


def _prep(x, au, bu, ad, bd):
    return pl.pallas_call(
        _prep_body,
        out_shape=[jax.ShapeDtypeStruct((N, H), jnp.float32)] * 2,
    )(x, au, bu.reshape(1, H), ad, bd.reshape(1, H))


def _t_body(attr_ref, w_ref, t_ref):
    t_ref[...] = jnp.dot(attr_ref[...], w_ref[...], preferred_element_type=jnp.float32)


def _edge_matmul(attr, w, block=1280):
    e = attr.shape[0]
    return pl.pallas_call(
        _t_body,
        grid=(e // block,),
        in_specs=[
            pl.BlockSpec((block, D), lambda i: (i, 0)),
            pl.BlockSpec((D, H), lambda i: (0, 0)),
        ],
        out_specs=pl.BlockSpec((block, H), lambda i: (i, 0)),
        out_shape=jax.ShapeDtypeStruct((e, H), jnp.float32),
    )(attr, w)


def _sc_body(yu, tu, u0, u1, yd, td, d0, d1, x, b0, b1,
             out_up, out_dn, out_b,
             acc, i0_v, i1_v, t_v, g_v, sem):
    c = lax.axis_index("c")
    s = lax.axis_index("s")

    def rows_copy(src, dst):
        r = pl.ds(pl.multiple_of(s * RPT, 8), RPT)
        pltpu.sync_copy(src.at[r], dst.at[r])

        @pl.when(s == 0)
        def _():
            rt = pl.ds(RPT * NS, RTAIL)
            pltpu.sync_copy(src.at[rt], dst.at[rt])

    def init_acc():
        rows_copy(x, acc)

    def edge_loop(y_ref, t_ref, i0_ref, i1_ref, edges_per_tile, with_t):
        n_chunks = edges_per_tile // C

        def chunk(j, carry):
            base = pl.multiple_of(s * edges_per_tile + j * C, 8)
            pltpu.sync_copy(i0_ref.at[pl.ds(base, C)], i0_v)
            pltpu.sync_copy(i1_ref.at[pl.ds(base, C)], i1_v)
            if with_t:
                pltpu.sync_copy(t_ref.at[pl.ds(base, C)], t_v)
            pltpu.async_copy(y_ref.at[i0_v], g_v, sem).wait()
            if with_t:
                def row(r, rcarry):
                    for cb in range(H // L):
                        sl = pl.ds(cb * L, L)
                        g_v[r, sl] = jnp.maximum(g_v[r, sl] + t_v[r, sl], 0.0)
                    return rcarry
                lax.fori_loop(0, C, row, 0, unroll=False)
            pltpu.sync_copy(g_v, acc.at[i1_v], add=True)
            return carry

        lax.fori_loop(0, n_chunks, chunk, 0, unroll=False)

    init_acc()
    plsc.subcore_barrier()

    @pl.when(c == 0)
    def _():
        edge_loop(yu, tu, u0, u1, E_UP // NS, True)

    @pl.when(c == 1)
    def _():
        edge_loop(yd, td, d0, d1, E_DOWN // NS, True)

    plsc.subcore_barrier()

    @pl.when(c == 0)
    def _():
        rows_copy(acc, out_up)

    @pl.when(c == 1)
    def _():
        rows_copy(acc, out_dn)

    plsc.subcore_barrier()

    @pl.when(c == 1)
    def _():
        init_acc()

    plsc.subcore_barrier()

    @pl.when(c == 1)
    def _():
        edge_loop(x, None, b0, b1, E_B // NS, False)

    plsc.subcore_barrier()

    @pl.when(c == 1)
    def _():
        rows_copy(acc, out_b)


def _sc_aggregate(yu, tu, u0, u1, yd, td, d0, d1, x, b0, b1):
    mesh = plsc.VectorSubcoreMesh(
        core_axis_name="c", subcore_axis_name="s", num_cores=NC, num_subcores=NS)
    return pl.kernel(
        _sc_body,
        out_type=[jax.ShapeDtypeStruct((N, H), jnp.float32)] * 3,
        mesh=mesh,
        scratch_types=[
            pltpu.VMEM_SHARED((N, H), jnp.float32),
            pltpu.VMEM((C,), jnp.int32),
            pltpu.VMEM((C,), jnp.int32),
            pltpu.VMEM((C, H), jnp.float32),
            pltpu.VMEM((C, H), jnp.float32),
            pltpu.SemaphoreType.DMA,
        ],
    )(yu, tu, u0, u1, yd, td, d0, d1, x, b0, b1)


def _bn_relu(h, g, b):
    mu = jnp.mean(h, axis=0, keepdims=True)
    var = jnp.mean((h - mu) * (h - mu), axis=0, keepdims=True)
    return jnp.maximum((h - mu) * lax.rsqrt(var + 1e-5) * g + b, 0.0)


def _update_path(h, w1, b1, g1, be1, w2, b2, g2, be2):
    h = _bn_relu(jnp.dot(h, w1, preferred_element_type=jnp.float32) + b1, g1, be1)
    h = _bn_relu(jnp.dot(h, w2, preferred_element_type=jnp.float32) + b2, g2, be2)
    return h


def _final_body(au_ref, ad_ref, ab_ref, *refs):
    prefs = refs[:-1]
    o_ref = refs[-1]
    pu = [r[...] for r in prefs[0:8]]
    pd = [r[...] for r in prefs[8:16]]
    pb = [r[...] for r in prefs[16:24]]
    wc1, wc2, wc3, bc, gc, bec = [r[...] for r in prefs[24:30]]
    hu = _update_path(au_ref[...], *pu)
    hd = _update_path(ad_ref[...], *pd)
    hb = _update_path(ab_ref[...], *pb)
    h = (jnp.dot(hu, wc1, preferred_element_type=jnp.float32)
         + jnp.dot(hd, wc2, preferred_element_type=jnp.float32)
         + jnp.dot(hb, wc3, preferred_element_type=jnp.float32)) + bc
    o_ref[...] = _bn_relu(h, gc, bec)


def _final(agg_u, agg_d, agg_b, pu, pd, pb, wc, bc, gc, bec):
    def flat(p):
        w1, b1, g1, be1, w2, b2, g2, be2 = p
        return [w1, b1.reshape(1, H), g1.reshape(1, H), be1.reshape(1, H),
                w2, b2.reshape(1, H), g2.reshape(1, H), be2.reshape(1, H)]

    args = ([agg_u, agg_d, agg_b] + flat(pu) + flat(pd) + flat(pb)
            + [wc[0:H], wc[H:2 * H], wc[2 * H:3 * H],
               bc.reshape(1, H), gc.reshape(1, H), bec.reshape(1, H)])
    return pl.pallas_call(
        _final_body,
        out_shape=jax.ShapeDtypeStruct((N, H), jnp.float32),
    )(*args)


@jax.jit
def kernel(x, up_index, up_attr, down_index, down_attr, boundary_index, params):
    wmu, bmu, wmd, bmd, pu, pd, pb, wc, bc, gc, bec = params
    yu, yd = _prep(x, wmu[:D], bmu, wmd[:D], bmd)
    tu = _edge_matmul(up_attr, wmu[D:])
    td = _edge_matmul(down_attr, wmd[D:])
    agg_u, agg_d, agg_b = _sc_aggregate(
        yu, tu, up_index[0], up_index[1],
        yd, td, down_index[0], down_index[1],
        x, boundary_index[0], boundary_index[1])
    return _final(agg_u, agg_d, agg_b, pu, pd, pb, wc, bc, gc, bec)
